# baseline (speedup 1.0000x reference)
import functools

import jax
import jax.numpy as jnp
from jax import lax
from jax.experimental import pallas as pl
from jax.experimental.pallas import tpu as pltpu
from jax.experimental.pallas import tpu_sc as plsc

B, L, WIN, H = 64, 4096, 5, 64
NUM_ST, NUM_IN = 16, 2
SIGMA = 0.5
NTAB = NUM_ST * NUM_IN
D = 2 * B
NW = 32
ROWS_PER_W = L // NW
CH = 512


@functools.lru_cache(maxsize=None)
def _make_sc_gather():
    mesh = plsc.VectorSubcoreMesh(core_axis_name="c", subcore_axis_name="s")

    @functools.partial(
        pl.kernel,
        out_type=jax.ShapeDtypeStruct((L, D), jnp.float32),
        mesh=mesh,
        scratch_types=[
            pltpu.VMEM((ROWS_PER_W,), jnp.int32),
            pltpu.VMEM((ROWS_PER_W, D), jnp.float32),
            pltpu.SemaphoreType.DMA,
        ],
    )
    def sc_gather(table_hbm, idx_hbm, out_hbm, idx_v, rows_v, sem):
        wid = lax.axis_index("s") * 2 + lax.axis_index("c")
        base = wid * ROWS_PER_W
        pltpu.sync_copy(idx_hbm.at[pl.ds(base, ROWS_PER_W)], idx_v)
        pltpu.async_copy(table_hbm.at[idx_v], rows_v, sem).wait()
        pltpu.sync_copy(rows_v, out_hbm.at[pl.ds(base, ROWS_PER_W)])

    return sc_gather


def _tree(masks, t):
    vals = [t[n] for n in range(NTAB)]
    for m in masks:
        vals = [jnp.where(m, vals[2 * j + 1], vals[2 * j])
                for j in range(len(vals) // 2)]
    return vals[0]


def _tc_body(bits, g, pi, w1a, b1a, w2a, b2a, w1b, b1b, w2b, b2b,
             o_par1, o_par2t, o_c1, o_c2):
    wb = 2.0 * pi[...] - 1.0

    def table(w1, b1, w2, b2):
        h = jnp.tanh(jnp.dot(wb, w1[...],
                             preferred_element_type=jnp.float32) + b1[...])
        t = jnp.tanh(jnp.dot(h, w2[...],
                             preferred_element_type=jnp.float32) + b2[...])
        return t[:, 0]

    ta = table(w1a, b1a, w2a, b2a)
    tb = table(w1b, b1b, w2b, b2b)

    bits_i = bits[...]
    gv = g[...]

    sum1 = sum2 = sq1 = sq2 = jnp.float32(0.0)
    for c in range(L // CH):
        lo = c * CH
        masks_a = []
        for k in range(WIN):
            if lo - k < 0:
                sh = jnp.concatenate(
                    [jnp.zeros((B, k - lo), jnp.int32),
                     bits_i[:, : CH - (k - lo)]], axis=1)
            else:
                sh = bits_i[:, lo - k: lo - k + CH]
            masks_a.append(sh != 0)
        pa = _tree(masks_a, ta)
        o_par1[:, lo:lo + CH] = pa
        sum1 += jnp.sum(pa)
        sq1 += jnp.sum(pa * pa)
        masks_b = []
        for k in range(WIN):
            if lo - k < 0:
                sh = jnp.concatenate(
                    [jnp.zeros((k - lo, B), jnp.float32),
                     gv[: CH - (k - lo), :B]], axis=0)
            else:
                sh = gv[lo - k: lo - k + CH, :B]
            masks_b.append(sh != 0.0)
        pb = _tree(masks_b, tb)
        o_par2t[lo:lo + CH, :] = pb
        sum2 += jnp.sum(pb)
        sq2 += jnp.sum(pb * pb)

    inv_n = jnp.float32(1.0 / (B * L))
    m1 = sum1 * inv_n
    m2 = sum2 * inv_n
    is1 = lax.rsqrt(jnp.maximum(sq1 * inv_n - m1 * m1, 1e-30))
    is2 = lax.rsqrt(jnp.maximum(sq2 * inv_n - m2 * m2, 1e-30))

    for c in range(L // CH):
        lo = c * CH
        o_par1[:, lo:lo + CH] = (o_par1[:, lo:lo + CH] - m1) * is1
        o_par2t[lo:lo + CH, :] = (o_par2t[lo:lo + CH, :] - m2) * is2

    o_c1[...] = jnp.concatenate(
        [wb[:, WIN - 1:WIN], ((ta - m1) * is1)[:, None]], axis=1)
    o_c2[...] = jnp.concatenate(
        [wb[:, WIN - 1:WIN], ((tb - m2) * is2)[:, None]], axis=1)


def _tc_call(bits, g, pi, *weights):
    return pl.pallas_call(
        _tc_body,
        out_shape=[
            jax.ShapeDtypeStruct((B, L), jnp.float32),
            jax.ShapeDtypeStruct((L, B), jnp.float32),
            jax.ShapeDtypeStruct((NTAB, 2), jnp.float32),
            jax.ShapeDtypeStruct((NTAB, 2), jnp.float32),
        ],
    )(bits, g, pi, *weights)


def kernel(input_stream, permutation, W1a, b1a, W2a, b2a, W1b, b1b, W2b, b2b,
           noise_sys, noise_par1, noise_par2, possible_inputs, next_states,
           prev_states):
    bits_f = input_stream.astype(jnp.float32)
    ns = noise_sys[:, :, 0]
    sysc = 2.0 * bits_f - 1.0 + SIGMA * ns
    packed = jnp.concatenate([bits_f.T, sysc.T], axis=1)
    g = _make_sc_gather()(packed, permutation.astype(jnp.int32))
    par1n, par2nt, c1, c2 = _tc_call(
        input_stream.astype(jnp.int32), g, possible_inputs,
        W1a, b1a.reshape(1, H), W2a, b2a.reshape(1, 1),
        W1b, b1b.reshape(1, H), W2b, b2b.reshape(1, 1))
    o_sys = (2.0 * bits_f - 1.0)[:, :, None] + SIGMA * noise_sys
    o_isys = g[:, B:].T[:, :, None]
    o_par1 = par1n[:, :, None] + SIGMA * noise_par1
    o_par2 = par2nt.T[:, :, None] + SIGMA * noise_par2
    return (o_sys, o_par1, o_isys, o_par2,
            c1.reshape(NUM_ST, NUM_IN, 2), c2.reshape(NUM_ST, NUM_IN, 2))


# speedup vs baseline: 1.1919x; 1.1919x over previous
import functools

import jax
import jax.numpy as jnp
from jax import lax
from jax.experimental import pallas as pl
from jax.experimental.pallas import tpu as pltpu
from jax.experimental.pallas import tpu_sc as plsc

B, L, WIN, H = 64, 4096, 5, 64
NUM_ST, NUM_IN = 16, 2
SIGMA = 0.5
NTAB = NUM_ST * NUM_IN
D = 2 * B
NW = 32
ROWS_PER_W = L // NW
CH = 512


@functools.lru_cache(maxsize=None)
def _make_sc_gather():
    mesh = plsc.VectorSubcoreMesh(core_axis_name="c", subcore_axis_name="s")

    @functools.partial(
        pl.kernel,
        out_type=jax.ShapeDtypeStruct((L, D), jnp.float32),
        mesh=mesh,
        scratch_types=[
            pltpu.VMEM((ROWS_PER_W,), jnp.int32),
            pltpu.VMEM((ROWS_PER_W, D), jnp.float32),
            pltpu.SemaphoreType.DMA,
        ],
    )
    def sc_gather(table_hbm, idx_hbm, out_hbm, idx_v, rows_v, sem):
        wid = lax.axis_index("s") * 2 + lax.axis_index("c")
        base = wid * ROWS_PER_W
        pltpu.sync_copy(idx_hbm.at[pl.ds(base, ROWS_PER_W)], idx_v)
        pltpu.async_copy(table_hbm.at[idx_v], rows_v, sem).wait()
        pltpu.sync_copy(rows_v, out_hbm.at[pl.ds(base, ROWS_PER_W)])

    return sc_gather


def _tree(masks, t):
    vals = [t[n] for n in range(NTAB)]
    for m in masks:
        vals = [jnp.where(m, vals[2 * j + 1], vals[2 * j])
                for j in range(len(vals) // 2)]
    return vals[0]


def _shift_chunk(x, lo, k):
    if lo - k < 0:
        return jnp.concatenate(
            [jnp.zeros((B, k - lo), x.dtype), x[:, : CH - (k - lo)]], axis=1)
    return x[:, lo - k: lo - k + CH]


def _tc_body(bits, bp, nsp, pi, w1a, b1a, w2a, b2a, w1b, b1b, w2b, b2b,
             o_par1, o_par2, o_isys, o_c1, o_c2):
    wb = 2.0 * pi[...] - 1.0

    def table(w1, b1, w2, b2):
        h = jnp.tanh(jnp.dot(wb, w1[...],
                             preferred_element_type=jnp.float32) + b1[...])
        t = jnp.tanh(jnp.dot(h, w2[...],
                             preferred_element_type=jnp.float32) + b2[...])
        return t[:, 0]

    ta = table(w1a, b1a, w2a, b2a)
    tb = table(w1b, b1b, w2b, b2b)

    bits_i = bits[...]
    bpf = bp[...]

    sum1 = sum2 = sq1 = sq2 = jnp.float32(0.0)
    for c in range(L // CH):
        lo = c * CH
        pa = _tree([_shift_chunk(bits_i, lo, k) != 0 for k in range(WIN)], ta)
        pb = _tree([_shift_chunk(bpf, lo, k) != 0.0 for k in range(WIN)], tb)
        o_par1[:, lo:lo + CH] = pa
        o_par2[:, lo:lo + CH] = pb
        sum1 += jnp.sum(pa)
        sq1 += jnp.sum(pa * pa)
        sum2 += jnp.sum(pb)
        sq2 += jnp.sum(pb * pb)
        o_isys[:, lo:lo + CH] = (
            2.0 * bpf[:, lo:lo + CH] - 1.0
            + SIGMA * nsp[:, lo:lo + CH])

    inv_n = jnp.float32(1.0 / (B * L))
    m1 = sum1 * inv_n
    m2 = sum2 * inv_n
    is1 = lax.rsqrt(jnp.maximum(sq1 * inv_n - m1 * m1, 1e-30))
    is2 = lax.rsqrt(jnp.maximum(sq2 * inv_n - m2 * m2, 1e-30))

    for c in range(L // CH):
        lo = c * CH
        o_par1[:, lo:lo + CH] = (o_par1[:, lo:lo + CH] - m1) * is1
        o_par2[:, lo:lo + CH] = (o_par2[:, lo:lo + CH] - m2) * is2

    o_c1[...] = jnp.concatenate(
        [wb[:, WIN - 1:WIN], ((ta - m1) * is1)[:, None]], axis=1)
    o_c2[...] = jnp.concatenate(
        [wb[:, WIN - 1:WIN], ((tb - m2) * is2)[:, None]], axis=1)


def _tc_call(bits, bp, nsp, pi, *weights):
    return pl.pallas_call(
        _tc_body,
        out_shape=[
            jax.ShapeDtypeStruct((B, L), jnp.float32),
            jax.ShapeDtypeStruct((B, L), jnp.float32),
            jax.ShapeDtypeStruct((B, L), jnp.float32),
            jax.ShapeDtypeStruct((NTAB, 2), jnp.float32),
            jax.ShapeDtypeStruct((NTAB, 2), jnp.float32),
        ],
    )(bits, bp, nsp, pi, *weights)


def kernel(input_stream, permutation, W1a, b1a, W2a, b2a, W1b, b1b, W2b, b2b,
           noise_sys, noise_par1, noise_par2, possible_inputs, next_states,
           prev_states):
    bits_f = input_stream.astype(jnp.float32)
    ns = noise_sys[:, :, 0]
    packed = jnp.concatenate([bits_f.T, ns.T], axis=1)
    g = _make_sc_gather()(packed, permutation.astype(jnp.int32))
    bp = g[:, :B].T
    nsp = g[:, B:].T
    par1n, par2n, isys, c1, c2 = _tc_call(
        input_stream.astype(jnp.int32), bp, nsp, possible_inputs,
        W1a, b1a.reshape(1, H), W2a, b2a.reshape(1, 1),
        W1b, b1b.reshape(1, H), W2b, b2b.reshape(1, 1))
    o_sys = (2.0 * bits_f - 1.0)[:, :, None] + SIGMA * noise_sys
    o_par1 = par1n[:, :, None] + SIGMA * noise_par1
    o_par2 = par2n[:, :, None] + SIGMA * noise_par2
    return (o_sys, o_par1, isys[:, :, None], o_par2,
            c1.reshape(NUM_ST, NUM_IN, 2), c2.reshape(NUM_ST, NUM_IN, 2))


# speedup vs baseline: 1.5590x; 1.3080x over previous
import functools

import jax
import jax.numpy as jnp
from jax import lax
from jax.experimental import pallas as pl
from jax.experimental.pallas import tpu as pltpu
from jax.experimental.pallas import tpu_sc as plsc

B, L, WIN, H = 64, 4096, 5, 64
NUM_ST, NUM_IN = 16, 2
SIGMA = 0.5
NTAB = NUM_ST * NUM_IN
D = 2 * B
NW = 32
ROWS_PER_W = L // NW
CH = 512


@functools.lru_cache(maxsize=None)
def _make_sc_gather():
    mesh = plsc.VectorSubcoreMesh(core_axis_name="c", subcore_axis_name="s")

    @functools.partial(
        pl.kernel,
        out_type=jax.ShapeDtypeStruct((L, D), jnp.float32),
        mesh=mesh,
        scratch_types=[
            pltpu.VMEM((ROWS_PER_W,), jnp.int32),
            pltpu.VMEM((ROWS_PER_W, D), jnp.float32),
            pltpu.SemaphoreType.DMA,
        ],
    )
    def sc_gather(table_hbm, idx_hbm, out_hbm, idx_v, rows_v, sem):
        wid = lax.axis_index("s") * 2 + lax.axis_index("c")
        base = wid * ROWS_PER_W
        pltpu.sync_copy(idx_hbm.at[pl.ds(base, ROWS_PER_W)], idx_v)
        pltpu.async_copy(table_hbm.at[idx_v], rows_v, sem).wait()
        pltpu.sync_copy(rows_v, out_hbm.at[pl.ds(base, ROWS_PER_W)])

    return sc_gather


def _tree(masks, t):
    vals = [t[n] for n in range(NTAB)]
    for m in masks:
        vals = [jnp.where(m, vals[2 * j + 1], vals[2 * j])
                for j in range(len(vals) // 2)]
    return vals[0]


def _shift_chunk(x, lo, k):
    if lo - k < 0:
        return jnp.concatenate(
            [jnp.zeros((B, k - lo), x.dtype), x[:, : CH - (k - lo)]], axis=1)
    return x[:, lo - k: lo - k + CH]


def _tc_body(bits, bp, nsp, pi, w1a, b1a, w2a, b2a, w1b, b1b, w2b, b2b,
             o_par1, o_par2, o_isys, o_c1, o_c2):
    wb = 2.0 * pi[...] - 1.0

    def table(w1, b1, w2, b2):
        h = jnp.tanh(jnp.dot(wb, w1[...],
                             preferred_element_type=jnp.float32) + b1[...])
        t = jnp.tanh(jnp.dot(h, w2[...],
                             preferred_element_type=jnp.float32) + b2[...])
        return t[:, 0]

    ta = table(w1a, b1a, w2a, b2a)
    tb = table(w1b, b1b, w2b, b2b)

    bits_i = bits[...]
    bpf = bp[...]
    bpi = bpf.astype(jnp.int32)

    def widx(b):
        acc = b
        for k in range(1, WIN):
            acc = acc + (1 << k) * jnp.concatenate(
                [jnp.zeros((B, k), jnp.int32), b[:, : L - k]], axis=1)
        return acc

    ta_b = jnp.broadcast_to(ta[None, :], (B, NTAB))
    tb_b = jnp.broadcast_to(tb[None, :], (B, NTAB))
    pa = jnp.take_along_axis(ta_b, widx(bits_i), axis=1)
    pb = jnp.take_along_axis(tb_b, widx(bpi), axis=1)
    o_par1[...] = pa
    o_par2[...] = pb
    sum1 = jnp.sum(pa); sq1 = jnp.sum(pa * pa)
    sum2 = jnp.sum(pb); sq2 = jnp.sum(pb * pb)
    o_isys[...] = 2.0 * bpf - 1.0 + SIGMA * nsp[...]

    inv_n = jnp.float32(1.0 / (B * L))
    m1 = sum1 * inv_n
    m2 = sum2 * inv_n
    is1 = lax.rsqrt(jnp.maximum(sq1 * inv_n - m1 * m1, 1e-30))
    is2 = lax.rsqrt(jnp.maximum(sq2 * inv_n - m2 * m2, 1e-30))

    for c in range(L // CH):
        lo = c * CH
        o_par1[:, lo:lo + CH] = (o_par1[:, lo:lo + CH] - m1) * is1
        o_par2[:, lo:lo + CH] = (o_par2[:, lo:lo + CH] - m2) * is2

    o_c1[...] = jnp.concatenate(
        [wb[:, WIN - 1:WIN], ((ta - m1) * is1)[:, None]], axis=1)
    o_c2[...] = jnp.concatenate(
        [wb[:, WIN - 1:WIN], ((tb - m2) * is2)[:, None]], axis=1)


def _tc_call(bits, bp, nsp, pi, *weights):
    return pl.pallas_call(
        _tc_body,
        out_shape=[
            jax.ShapeDtypeStruct((B, L), jnp.float32),
            jax.ShapeDtypeStruct((B, L), jnp.float32),
            jax.ShapeDtypeStruct((B, L), jnp.float32),
            jax.ShapeDtypeStruct((NTAB, 2), jnp.float32),
            jax.ShapeDtypeStruct((NTAB, 2), jnp.float32),
        ],
    )(bits, bp, nsp, pi, *weights)


def kernel(input_stream, permutation, W1a, b1a, W2a, b2a, W1b, b1b, W2b, b2b,
           noise_sys, noise_par1, noise_par2, possible_inputs, next_states,
           prev_states):
    bits_f = input_stream.astype(jnp.float32)
    ns = noise_sys[:, :, 0]
    packed = jnp.concatenate([bits_f.T, ns.T], axis=1)
    g = _make_sc_gather()(packed, permutation.astype(jnp.int32))
    bp = g[:, :B].T
    nsp = g[:, B:].T
    par1n, par2n, isys, c1, c2 = _tc_call(
        input_stream.astype(jnp.int32), bp, nsp, possible_inputs,
        W1a, b1a.reshape(1, H), W2a, b2a.reshape(1, 1),
        W1b, b1b.reshape(1, H), W2b, b2b.reshape(1, 1))
    o_sys = (2.0 * bits_f - 1.0)[:, :, None] + SIGMA * noise_sys
    o_par1 = par1n[:, :, None] + SIGMA * noise_par1
    o_par2 = par2n[:, :, None] + SIGMA * noise_par2
    return (o_sys, o_par1, isys[:, :, None], o_par2,
            c1.reshape(NUM_ST, NUM_IN, 2), c2.reshape(NUM_ST, NUM_IN, 2))


# speedup vs baseline: 1.7640x; 1.1315x over previous
import functools

import jax
import jax.numpy as jnp
from jax import lax
from jax.experimental import pallas as pl
from jax.experimental.pallas import tpu as pltpu
from jax.experimental.pallas import tpu_sc as plsc

B, L, WIN, H = 64, 4096, 5, 64
NUM_ST, NUM_IN = 16, 2
SIGMA = 0.5
NTAB = NUM_ST * NUM_IN
D = 2 * B
NW = 32
ROWS_PER_W = L // NW
CH = 512


@functools.lru_cache(maxsize=None)
def _make_sc_gather():
    mesh = plsc.VectorSubcoreMesh(core_axis_name="c", subcore_axis_name="s")

    @functools.partial(
        pl.kernel,
        out_type=jax.ShapeDtypeStruct((L, D), jnp.float32),
        mesh=mesh,
        scratch_types=[
            pltpu.VMEM((ROWS_PER_W,), jnp.int32),
            pltpu.VMEM((ROWS_PER_W, D), jnp.float32),
            pltpu.SemaphoreType.DMA,
        ],
    )
    def sc_gather(table_hbm, idx_hbm, out_hbm, idx_v, rows_v, sem):
        wid = lax.axis_index("s") * 2 + lax.axis_index("c")
        base = wid * ROWS_PER_W
        pltpu.sync_copy(idx_hbm.at[pl.ds(base, ROWS_PER_W)], idx_v)
        pltpu.async_copy(table_hbm.at[idx_v], rows_v, sem).wait()
        pltpu.sync_copy(rows_v, out_hbm.at[pl.ds(base, ROWS_PER_W)])

    return sc_gather


def _tree(masks, t):
    vals = [t[n] for n in range(NTAB)]
    for m in masks:
        vals = [jnp.where(m, vals[2 * j + 1], vals[2 * j])
                for j in range(len(vals) // 2)]
    return vals[0]


def _shift_chunk(x, lo, k):
    if lo - k < 0:
        return jnp.concatenate(
            [jnp.zeros((B, k - lo), x.dtype), x[:, : CH - (k - lo)]], axis=1)
    return x[:, lo - k: lo - k + CH]


def _tc_body(bits, g, pi, w1a, b1a, w2a, b2a, w1b, b1b, w2b, b2b,
             o_par1, o_par2, o_isys, o_c1, o_c2):
    wb = 2.0 * pi[...] - 1.0

    def table(w1, b1, w2, b2):
        h = jnp.tanh(jnp.dot(wb, w1[...],
                             preferred_element_type=jnp.float32) + b1[...])
        t = jnp.tanh(jnp.dot(h, w2[...],
                             preferred_element_type=jnp.float32) + b2[...])
        return t[:, 0]

    ta = table(w1a, b1a, w2a, b2a)
    tb = table(w1b, b1b, w2b, b2b)

    bits_i = bits[...]
    gv = g[...]
    eye = (lax.broadcasted_iota(jnp.int32, (B, B), 0)
           == lax.broadcasted_iota(jnp.int32, (B, B), 1)).astype(jnp.float32)
    tdn = (((1,), (1,)), ((), ()))
    bpf = lax.dot_general(eye, gv[:, :B], tdn,
                          preferred_element_type=jnp.float32)
    nspv = lax.dot_general(eye, gv[:, B:], tdn,
                           preferred_element_type=jnp.float32)
    bpi = bpf.astype(jnp.int32)

    def widx(b):
        acc = b
        for k in range(1, WIN):
            acc = acc + (1 << k) * jnp.concatenate(
                [jnp.zeros((B, k), jnp.int32), b[:, : L - k]], axis=1)
        return acc

    ta_b = jnp.broadcast_to(ta[None, :], (B, NTAB))
    tb_b = jnp.broadcast_to(tb[None, :], (B, NTAB))
    pa = jnp.take_along_axis(ta_b, widx(bits_i), axis=1)
    pb = jnp.take_along_axis(tb_b, widx(bpi), axis=1)
    o_par1[...] = pa
    o_par2[...] = pb
    sum1 = jnp.sum(pa); sq1 = jnp.sum(pa * pa)
    sum2 = jnp.sum(pb); sq2 = jnp.sum(pb * pb)
    o_isys[...] = 2.0 * bpf - 1.0 + SIGMA * nspv

    inv_n = jnp.float32(1.0 / (B * L))
    m1 = sum1 * inv_n
    m2 = sum2 * inv_n
    is1 = lax.rsqrt(jnp.maximum(sq1 * inv_n - m1 * m1, 1e-30))
    is2 = lax.rsqrt(jnp.maximum(sq2 * inv_n - m2 * m2, 1e-30))

    for c in range(L // CH):
        lo = c * CH
        o_par1[:, lo:lo + CH] = (o_par1[:, lo:lo + CH] - m1) * is1
        o_par2[:, lo:lo + CH] = (o_par2[:, lo:lo + CH] - m2) * is2

    o_c1[...] = jnp.concatenate(
        [wb[:, WIN - 1:WIN], ((ta - m1) * is1)[:, None]], axis=1)
    o_c2[...] = jnp.concatenate(
        [wb[:, WIN - 1:WIN], ((tb - m2) * is2)[:, None]], axis=1)


def _tc_call(bits, g, pi, *weights):
    return pl.pallas_call(
        _tc_body,
        out_shape=[
            jax.ShapeDtypeStruct((B, L), jnp.float32),
            jax.ShapeDtypeStruct((B, L), jnp.float32),
            jax.ShapeDtypeStruct((B, L), jnp.float32),
            jax.ShapeDtypeStruct((NTAB, 2), jnp.float32),
            jax.ShapeDtypeStruct((NTAB, 2), jnp.float32),
        ],
    )(bits, g, pi, *weights)


def kernel(input_stream, permutation, W1a, b1a, W2a, b2a, W1b, b1b, W2b, b2b,
           noise_sys, noise_par1, noise_par2, possible_inputs, next_states,
           prev_states):
    bits_f = input_stream.astype(jnp.float32)
    ns = noise_sys[:, :, 0]
    packed = jnp.concatenate([bits_f.T, ns.T], axis=1)
    g = _make_sc_gather()(packed, permutation.astype(jnp.int32))
    par1n, par2n, isys, c1, c2 = _tc_call(
        input_stream.astype(jnp.int32), g, possible_inputs,
        W1a, b1a.reshape(1, H), W2a, b2a.reshape(1, 1),
        W1b, b1b.reshape(1, H), W2b, b2b.reshape(1, 1))
    o_sys = (2.0 * bits_f - 1.0)[:, :, None] + SIGMA * noise_sys
    o_par1 = par1n[:, :, None] + SIGMA * noise_par1
    o_par2 = par2n[:, :, None] + SIGMA * noise_par2
    return (o_sys, o_par1, isys[:, :, None], o_par2,
            c1.reshape(NUM_ST, NUM_IN, 2), c2.reshape(NUM_ST, NUM_IN, 2))
